# baseline (device time: 151476 ns/iter reference)
import jax
import jax.numpy as jnp
from jax import lax
from jax.experimental import pallas as pl
from jax.experimental.pallas import tpu as pltpu

N_DEV = 4
N_HOP = 2 * (N_DEV - 1)
N_SUB = 2


def kernel(x):
    m_per, n = x.shape
    half = m_per // 2
    chunk = half // N_DEV
    sub = chunk // N_SUB

    def send_chunk(d, t, ring):
        if t < N_DEV - 1:
            return (d - t) % N_DEV if ring == "a" else (d + t) % N_DEV
        s = t - (N_DEV - 1)
        return (d + 1 - s) % N_DEV if ring == "a" else (d - 1 + s) % N_DEV

    def recv_chunk(d, t, ring):
        if t < N_DEV - 1:
            return (d - t - 1) % N_DEV if ring == "a" else (d + t + 1) % N_DEV
        s = t - (N_DEV - 1)
        return (d - s) % N_DEV if ring == "a" else (d + s) % N_DEV

    def body(x_ref, out_ref, comm_a, comm_b,
             send_sems_a, recv_sems_a, send_sems_b, recv_sems_b):
        d = lax.axis_index("i")
        left = (d - 1) % N_DEV
        right = (d + 1) % N_DEV

        barrier_sem = pltpu.get_barrier_semaphore()
        for nbr in (left, right):
            pl.semaphore_signal(
                barrier_sem, inc=1,
                device_id=(nbr,), device_id_type=pl.DeviceIdType.MESH,
            )
        pl.semaphore_wait(barrier_sem, 2)

        def rows(t, s, ring, which):
            c = (send_chunk if which == "send" else recv_chunk)(d, t, ring)
            base = 0 if ring == "a" else half
            return pl.ds(base + c * chunk + s * sub, sub)

        def ring_parts(ring):
            if ring == "a":
                return comm_a, send_sems_a, recv_sems_a, right, left
            return comm_b, send_sems_b, recv_sems_b, left, right

        def make_send(t, s, ring):
            comm, send_sems, recv_sems, dst_nbr, _ = ring_parts(ring)
            r = rows(t, s, ring, "send")
            src = x_ref.at[r, :] if t == 0 else out_ref.at[r, :]
            dst = comm.at[t, s] if t < N_DEV - 1 else out_ref.at[r, :]
            return pltpu.make_async_remote_copy(
                src_ref=src, dst_ref=dst,
                send_sem=send_sems.at[t, s], recv_sem=recv_sems.at[t, s],
                device_id=(dst_nbr,), device_id_type=pl.DeviceIdType.MESH,
            )

        pending = []
        for s in range(N_SUB):
            for ring in ("a", "b"):
                r = make_send(0, s, ring)
                r.start()
                pending.append(r)

        for t in range(N_HOP):
            for s in range(N_SUB):
                for ring in ("a", "b"):
                    comm, _, recv_sems, _, src_nbr = ring_parts(ring)
                    rr = rows(t, s, ring, "recv")
                    dst = comm.at[t, s] if t < N_DEV - 1 else out_ref.at[rr, :]
                    recv = pltpu.make_async_remote_copy(
                        src_ref=dst, dst_ref=dst,
                        send_sem=recv_sems.at[t, s], recv_sem=recv_sems.at[t, s],
                        device_id=(src_nbr,), device_id_type=pl.DeviceIdType.MESH,
                    )
                    recv.wait_recv()
                    if t < N_DEV - 1:
                        out_ref[rr, :] = x_ref[rr, :] + comm[t, s]
                    if t < N_HOP - 1:
                        r = make_send(t + 1, s, ring)
                        r.start()
                        pending.append(r)

        for r in pending:
            r.wait_send()

    return pl.pallas_call(
        body,
        out_shape=jax.ShapeDtypeStruct((m_per, n), x.dtype),
        in_specs=[pl.BlockSpec(memory_space=pltpu.VMEM)],
        out_specs=pl.BlockSpec(memory_space=pltpu.VMEM),
        scratch_shapes=[
            pltpu.VMEM((N_DEV - 1, N_SUB, sub, n), x.dtype),
            pltpu.VMEM((N_DEV - 1, N_SUB, sub, n), x.dtype),
            pltpu.SemaphoreType.DMA((N_HOP, N_SUB)),
            pltpu.SemaphoreType.DMA((N_HOP, N_SUB)),
            pltpu.SemaphoreType.DMA((N_HOP, N_SUB)),
            pltpu.SemaphoreType.DMA((N_HOP, N_SUB)),
        ],
        compiler_params=pltpu.CompilerParams(collective_id=0),
    )(x)


# device time: 151307 ns/iter; 1.0011x vs baseline; 1.0011x over previous
import jax
import jax.numpy as jnp
from jax import lax
from jax.experimental import pallas as pl
from jax.experimental.pallas import tpu as pltpu

N_DEV = 4
N_HOP = 2 * (N_DEV - 1)
N_SUB = 4


def kernel(x):
    m_per, n = x.shape
    half = m_per // 2
    chunk = half // N_DEV
    sub = chunk // N_SUB

    def send_chunk(d, t, ring):
        if t < N_DEV - 1:
            return (d - t) % N_DEV if ring == "a" else (d + t) % N_DEV
        s = t - (N_DEV - 1)
        return (d + 1 - s) % N_DEV if ring == "a" else (d - 1 + s) % N_DEV

    def recv_chunk(d, t, ring):
        if t < N_DEV - 1:
            return (d - t - 1) % N_DEV if ring == "a" else (d + t + 1) % N_DEV
        s = t - (N_DEV - 1)
        return (d - s) % N_DEV if ring == "a" else (d + s) % N_DEV

    def body(x_ref, out_ref, comm_a, comm_b,
             send_sems_a, recv_sems_a, send_sems_b, recv_sems_b):
        d = lax.axis_index("i")
        left = (d - 1) % N_DEV
        right = (d + 1) % N_DEV

        barrier_sem = pltpu.get_barrier_semaphore()
        for nbr in (left, right):
            pl.semaphore_signal(
                barrier_sem, inc=1,
                device_id=(nbr,), device_id_type=pl.DeviceIdType.MESH,
            )
        pl.semaphore_wait(barrier_sem, 2)

        def rows(t, s, ring, which):
            c = (send_chunk if which == "send" else recv_chunk)(d, t, ring)
            base = 0 if ring == "a" else half
            return pl.ds(base + c * chunk + s * sub, sub)

        def ring_parts(ring):
            if ring == "a":
                return comm_a, send_sems_a, recv_sems_a, right, left
            return comm_b, send_sems_b, recv_sems_b, left, right

        def make_send(t, s, ring):
            comm, send_sems, recv_sems, dst_nbr, _ = ring_parts(ring)
            r = rows(t, s, ring, "send")
            src = x_ref.at[r, :] if t == 0 else out_ref.at[r, :]
            dst = comm.at[t, s] if t < N_DEV - 1 else out_ref.at[r, :]
            return pltpu.make_async_remote_copy(
                src_ref=src, dst_ref=dst,
                send_sem=send_sems.at[t, s], recv_sem=recv_sems.at[t, s],
                device_id=(dst_nbr,), device_id_type=pl.DeviceIdType.MESH,
            )

        pending = []
        for s in range(N_SUB):
            for ring in ("a", "b"):
                r = make_send(0, s, ring)
                r.start()
                pending.append(r)

        for t in range(N_HOP):
            for s in range(N_SUB):
                for ring in ("a", "b"):
                    comm, _, recv_sems, _, src_nbr = ring_parts(ring)
                    rr = rows(t, s, ring, "recv")
                    dst = comm.at[t, s] if t < N_DEV - 1 else out_ref.at[rr, :]
                    recv = pltpu.make_async_remote_copy(
                        src_ref=dst, dst_ref=dst,
                        send_sem=recv_sems.at[t, s], recv_sem=recv_sems.at[t, s],
                        device_id=(src_nbr,), device_id_type=pl.DeviceIdType.MESH,
                    )
                    recv.wait_recv()
                    if t < N_DEV - 1:
                        out_ref[rr, :] = x_ref[rr, :] + comm[t, s]
                    if t < N_HOP - 1:
                        r = make_send(t + 1, s, ring)
                        r.start()
                        pending.append(r)

        for r in pending:
            r.wait_send()

    return pl.pallas_call(
        body,
        out_shape=jax.ShapeDtypeStruct((m_per, n), x.dtype),
        in_specs=[pl.BlockSpec(memory_space=pltpu.VMEM)],
        out_specs=pl.BlockSpec(memory_space=pltpu.VMEM),
        scratch_shapes=[
            pltpu.VMEM((N_DEV - 1, N_SUB, sub, n), x.dtype),
            pltpu.VMEM((N_DEV - 1, N_SUB, sub, n), x.dtype),
            pltpu.SemaphoreType.DMA((N_HOP, N_SUB)),
            pltpu.SemaphoreType.DMA((N_HOP, N_SUB)),
            pltpu.SemaphoreType.DMA((N_HOP, N_SUB)),
            pltpu.SemaphoreType.DMA((N_HOP, N_SUB)),
        ],
        compiler_params=pltpu.CompilerParams(collective_id=0),
    )(x)
